# Initial kernel scaffold; baseline (speedup 1.0000x reference)
import jax
import jax.numpy as jnp
from jax.experimental import pallas as pl


def kernel(mem_input, mem, ts, mem_ts, h, time_w, time_b, weight_ih, weight_hh, bias, node_W, node_b):
    raise NotImplementedError("write your pallas kernel here")


# speedup vs baseline: 1.4028x; 1.4028x over previous
import jax
import jax.numpy as jnp
from jax.experimental import pallas as pl
from jax.experimental.pallas import tpu as pltpu

_MIN_NORM = 1e-15
_BALL_EPS = 4e-3


def _rowsum(x):
    return jnp.sum(x, axis=-1, keepdims=True)


def _artanh(x):
    x = jnp.clip(x, -1 + 1e-7, 1 - 1e-7)
    return 0.5 * (jnp.log1p(x) - jnp.log1p(-x))


def _gru_body(mi_ref, mem_ref, ts_ref, mts_ref, h_ref, tw_ref, tb_ref,
              wih_ref, whh_ref, bias_ref, nw_ref, nb_ref, out_ref):
    f32 = jnp.float32
    mi = mi_ref[...]
    hx = mem_ref[...]

    dt = ts_ref[...] - mts_ref[...]
    ang = dt * tw_ref[...] + tb_ref[...]
    u = jnp.cos(ang)
    un = jnp.maximum(jnp.sqrt(_rowsum(u * u)), _MIN_NORM)
    tf = (jnp.tanh(un) / un) * u
    tfn = jnp.maximum(jnp.sqrt(_rowsum(tf * tf)), _MIN_NORM)
    maxnorm = 1.0 - _BALL_EPS
    proj_scale = jnp.where(tfn > maxnorm, maxnorm / tfn, 1.0)
    tf = proj_scale * tf

    mi2 = _rowsum(mi * mi)
    tf2 = _rowsum(tf * tf)
    xn = jnp.maximum(jnp.sqrt(mi2 + tf2), _MIN_NORM)
    hx2 = _rowsum(hx * hx)
    hn = jnp.maximum(jnp.sqrt(hx2), _MIN_NORM)
    aox = _artanh(xn) / xn
    aoh = _artanh(hn) / hn

    x_cat = jnp.concatenate([mi, tf], axis=1)
    ux_all = jnp.dot(x_cat, wih_ref[...], preferred_element_type=f32)
    wh_all = jnp.dot(hx, whh_ref[...], preferred_element_type=f32)

    def mmv_post(m, a_over_n):
        mxn = jnp.maximum(jnp.sqrt(_rowsum(m * m)), _MIN_NORM)
        return (jnp.tanh(a_over_n * mxn) / mxn) * m

    H = hx.shape[-1]
    ux_r = mmv_post(ux_all[:, 0:H], aox)
    ux_h = mmv_post(ux_all[:, H:2 * H], aox)
    ux_z = mmv_post(ux_all[:, 2 * H:3 * H], aox)
    wh_r = mmv_post(wh_all[:, 0:H], aoh)
    wh_z = mmv_post(wh_all[:, 2 * H:3 * H], aoh)

    def madd(x, y, x2=None, y2=None):
        if x2 is None:
            x2 = _rowsum(x * x)
        if y2 is None:
            y2 = _rowsum(y * y)
        xy = _rowsum(x * y)
        num = (1.0 + 2.0 * xy + y2) * x + (1.0 - x2) * y
        den = jnp.maximum(1.0 + 2.0 * xy + x2 * y2, _MIN_NORM)
        return num / den

    b_r = bias_ref[0:1, :]
    b_h = bias_ref[1:2, :]
    b_z = bias_ref[2:3, :]

    gz = madd(madd(wh_z, ux_z), b_z)
    gr = madd(madd(wh_r, ux_r), b_r)

    def logmap_sig(y):
        n = jnp.maximum(jnp.sqrt(_rowsum(y * y)), _MIN_NORM)
        return jax.nn.sigmoid((_artanh(n) / n) * y)

    z = logmap_sig(gz)
    r = logmap_sig(gr)

    def mpm(w, x, a_over_n):
        wx = w * x
        wxn = jnp.maximum(jnp.sqrt(_rowsum(wx * wx)), _MIN_NORM)
        return (jnp.tanh(a_over_n * wxn) / wxn) * wx

    rh = mpm(r, hx, aoh)
    rhn = jnp.maximum(jnp.sqrt(_rowsum(rh * rh)), _MIN_NORM)
    aorh = _artanh(rhn) / rhn
    wh_h = jnp.dot(rh, whh_ref[:, H:2 * H], preferred_element_type=f32)
    wh_h = mmv_post(wh_h, aorh)
    h_tilde = madd(madd(wh_h, ux_h), b_h)

    delta = madd(-hx, h_tilde, x2=hx2)
    dn = jnp.maximum(jnp.sqrt(_rowsum(delta * delta)), _MIN_NORM)
    aod = _artanh(dn) / dn
    zd = mpm(z, delta, aod)
    upd = madd(hx, zd, x2=hx2)

    hm = jnp.dot(h_ref[...], nw_ref[...], preferred_element_type=f32) + nb_ref[...]
    out_ref[...] = madd(upd, hm)


def kernel(mem_input, mem, ts, mem_ts, h, time_w, time_b,
           weight_ih, weight_hh, bias, node_W, node_b):
    N, D_in = mem_input.shape
    H = mem.shape[1]
    D_node = h.shape[1]
    D_t = time_w.shape[0]

    B = 512
    grid = (N // B,)

    wih_t = weight_ih.T
    whh_t = weight_hh.T
    nw_t = node_W.T
    ts2 = ts[:, None]
    mts2 = mem_ts[:, None]
    tw = time_w[None, :]
    tb = time_b[None, :]
    nb = node_b[None, :]

    fixed = lambda i: (0, 0)
    rows = lambda i: (i, 0)

    return pl.pallas_call(
        _gru_body,
        grid=grid,
        in_specs=[
            pl.BlockSpec((B, D_in), rows),
            pl.BlockSpec((B, H), rows),
            pl.BlockSpec((B, 1), rows),
            pl.BlockSpec((B, 1), rows),
            pl.BlockSpec((B, D_node), rows),
            pl.BlockSpec((1, D_t), fixed),
            pl.BlockSpec((1, D_t), fixed),
            pl.BlockSpec((D_in + D_t, 3 * H), fixed),
            pl.BlockSpec((H, 3 * H), fixed),
            pl.BlockSpec((3, H), fixed),
            pl.BlockSpec((D_node, H), fixed),
            pl.BlockSpec((1, H), fixed),
        ],
        out_specs=pl.BlockSpec((B, H), rows),
        out_shape=jax.ShapeDtypeStruct((N, H), jnp.float32),
        compiler_params=pltpu.CompilerParams(
            dimension_semantics=("parallel",),
        ),
    )(mem_input, mem, ts2, mts2, h, tw, tb, wih_t, whh_t, bias, nw_t, nb)


# speedup vs baseline: 2.0747x; 1.4789x over previous
import jax
import jax.numpy as jnp
from jax.experimental import pallas as pl
from jax.experimental.pallas import tpu as pltpu

_MIN_NORM = 1e-15
_BALL_EPS = 4e-3
_HALF_LN2 = 0.34657359027997264

_INV_PIO2 = 0.6366197723675814
_DP1 = 1.5703125
_DP2 = 4.837512969970703125e-4
_DP3 = 7.549789948768648e-8


def _rowsum(x):
    return jnp.sum(x, axis=-1, keepdims=True)


def _fast_cos(x):
    qi = jnp.round(x * _INV_PIO2).astype(jnp.int32)
    qf = qi.astype(jnp.float32)
    r = x - qf * _DP1
    r = r - qf * _DP2
    r = r - qf * _DP3
    x2 = r * r
    cosp = ((2.443315711809948e-5 * x2 - 1.388731625493765e-3) * x2
            + 4.166664568298827e-2) * (x2 * x2) + (1.0 - 0.5 * x2)
    sinp = (((-1.9515295891e-4 * x2 + 8.3321608736e-3) * x2
             - 1.6666654611e-1) * x2) * r + r
    val = jnp.where((qi & 1) == 1, sinp, cosp)
    sign = ((qi + 1) & 2) << 30
    bits = pltpu.bitcast(val, jnp.int32) ^ sign
    return pltpu.bitcast(bits, jnp.float32)


def _norm_inv(x):
    ss = jnp.maximum(_rowsum(x * x), _MIN_NORM * _MIN_NORM)
    rn = jax.lax.rsqrt(ss)
    return ss, ss * rn, rn


def _aon(n, inv_n):
    nc = jnp.minimum(n, 1.0 - 1e-7)
    t = (1.0 + nc) / (1.0 - nc)
    return (_HALF_LN2 * inv_n) * jnp.log2(t)


def _gru_body(mi_ref, mem_ref, ts_ref, mts_ref, h_ref, tw_ref, tb_ref,
              wih_ref, whh_ref, bias_ref, nw_ref, nb_ref, out_ref):
    f32 = jnp.float32
    mi = mi_ref[...]
    hx = mem_ref[...]
    H = hx.shape[-1]

    dt = ts_ref[...] - mts_ref[...]
    ang = jnp.dot(dt, tw_ref[...], preferred_element_type=f32) + tb_ref[...]
    u = _fast_cos(ang)
    _, un, inv_un = _norm_inv(u)
    tn = jnp.tanh(un)
    maxnorm = 1.0 - _BALL_EPS
    pscale = jnp.where(tn > maxnorm, maxnorm / tn, 1.0)
    tf = (tn * inv_un * pscale) * u
    tfn = jnp.minimum(tn, maxnorm)

    mi2 = _rowsum(mi * mi)
    xss = jnp.maximum(mi2 + tfn * tfn, _MIN_NORM * _MIN_NORM)
    inv_xn = jax.lax.rsqrt(xss)
    xn = xss * inv_xn
    hss, hn, inv_hn = _norm_inv(hx)
    aox = _aon(xn, inv_xn)
    aoh = _aon(hn, inv_hn)

    x_cat = jnp.concatenate([mi, tf], axis=1)
    ux_all = jnp.dot(x_cat, wih_ref[...], preferred_element_type=f32)
    wh_all = jnp.dot(hx, whh_ref[...], preferred_element_type=f32)

    def mmv_post(m, aon_src):
        _, mxn, inv_mxn = _norm_inv(m)
        t = jnp.tanh(aon_src * mxn)
        return (t * inv_mxn) * m, t

    ux_r, t_uxr = mmv_post(ux_all[:, 0:H], aox)
    ux_h, t_uxh = mmv_post(ux_all[:, H:2 * H], aox)
    ux_z, t_uxz = mmv_post(ux_all[:, 2 * H:3 * H], aox)
    wh_r, t_whr = mmv_post(wh_all[:, 0:H], aoh)
    wh_z, t_whz = mmv_post(wh_all[:, 2 * H:3 * H], aoh)

    def madd(x, y, x2=None, y2=None):
        if x2 is None:
            x2 = _rowsum(x * x)
        if y2 is None:
            y2 = _rowsum(y * y)
        xy = _rowsum(x * y)
        num = (1.0 + 2.0 * xy + y2) * x + (1.0 - x2) * y
        inv_den = 1.0 / jnp.maximum(1.0 + 2.0 * xy + x2 * y2, _MIN_NORM)
        return num * inv_den

    b_r = bias_ref[0:1, :]
    b_h = bias_ref[1:2, :]
    b_z = bias_ref[2:3, :]
    b_r2 = _rowsum(b_r * b_r)
    b_h2 = _rowsum(b_h * b_h)
    b_z2 = _rowsum(b_z * b_z)

    gz = madd(madd(wh_z, ux_z, x2=t_whz * t_whz, y2=t_uxz * t_uxz), b_z, y2=b_z2)
    gr = madd(madd(wh_r, ux_r, x2=t_whr * t_whr, y2=t_uxr * t_uxr), b_r, y2=b_r2)

    def logmap_sig(y):
        _, n, inv_n = _norm_inv(y)
        return jax.nn.sigmoid(_aon(n, inv_n) * y)

    z = logmap_sig(gz)
    r = logmap_sig(gr)

    def mpm(w, x, aon_x):
        wx = w * x
        _, wxn, inv_wxn = _norm_inv(wx)
        t = jnp.tanh(aon_x * wxn)
        return (t * inv_wxn) * wx, t

    rh, t_rh = mpm(r, hx, aoh)
    rhn = jnp.maximum(t_rh, _MIN_NORM)
    aorh = _aon(rhn, 1.0 / rhn)
    wh_h = jnp.dot(rh, whh_ref[:, H:2 * H], preferred_element_type=f32)
    wh_h, t_whh = mmv_post(wh_h, aorh)
    h_tilde = madd(madd(wh_h, ux_h, x2=t_whh * t_whh, y2=t_uxh * t_uxh), b_h, y2=b_h2)

    delta = madd(-hx, h_tilde, x2=hss)
    _, dn, inv_dn = _norm_inv(delta)
    aod = _aon(dn, inv_dn)
    zd, t_zd = mpm(z, delta, aod)
    upd = madd(hx, zd, x2=hss, y2=t_zd * t_zd)

    hm = jnp.dot(h_ref[...], nw_ref[...], preferred_element_type=f32) + nb_ref[...]
    out_ref[...] = madd(upd, hm)


def kernel(mem_input, mem, ts, mem_ts, h, time_w, time_b,
           weight_ih, weight_hh, bias, node_W, node_b):
    N, D_in = mem_input.shape
    H = mem.shape[1]
    D_node = h.shape[1]
    D_t = time_w.shape[0]

    B = 512
    grid = (N // B,)

    wih_t = weight_ih.T
    whh_t = weight_hh.T
    nw_t = node_W.T
    ts2 = ts[:, None]
    mts2 = mem_ts[:, None]
    tw = time_w[None, :]
    tb = time_b[None, :]
    nb = node_b[None, :]

    fixed = lambda i: (0, 0)
    rows = lambda i: (i, 0)

    return pl.pallas_call(
        _gru_body,
        grid=grid,
        in_specs=[
            pl.BlockSpec((B, D_in), rows),
            pl.BlockSpec((B, H), rows),
            pl.BlockSpec((B, 1), rows),
            pl.BlockSpec((B, 1), rows),
            pl.BlockSpec((B, D_node), rows),
            pl.BlockSpec((1, D_t), fixed),
            pl.BlockSpec((1, D_t), fixed),
            pl.BlockSpec((D_in + D_t, 3 * H), fixed),
            pl.BlockSpec((H, 3 * H), fixed),
            pl.BlockSpec((3, H), fixed),
            pl.BlockSpec((D_node, H), fixed),
            pl.BlockSpec((1, H), fixed),
        ],
        out_specs=pl.BlockSpec((B, H), rows),
        out_shape=jax.ShapeDtypeStruct((N, H), jnp.float32),
        compiler_params=pltpu.CompilerParams(
            dimension_semantics=("parallel",),
        ),
    )(mem_input, mem, ts2, mts2, h, tw, tb, wih_t, whh_t, bias, nw_t, nb)
